# baseline (speedup 1.0000x reference)
import functools

import jax
import jax.numpy as jnp
import numpy as np
from jax.experimental import pallas as pl
from jax.experimental.pallas import tpu as pltpu

N_NODES = 10000
HID = 64
_NFULL = N_NODES // 128
_NTAIL = N_NODES - _NFULL * 128
_INV_N = float(np.float32(1e-4))
_EPS = float(np.float32(1e-5))


def _sum_nodes(h):
    acc = h[:, 0:128]
    for j in range(1, _NFULL):
        acc = acc + h[:, 128 * j:128 * (j + 1)]
    tail = h[:, _NFULL * 128:N_NODES]
    acc = jnp.concatenate([acc[:, 0:_NTAIL] + tail, acc[:, _NTAIL:]], axis=1)
    g = acc[:, 0:8]
    for k in range(1, 16):
        g = g + acc[:, 8 * k:8 * k + 8]
    g = g[:, 0:4] + g[:, 4:8]
    g = g[:, 0:2] + g[:, 2:4]
    return g[:, 0:1] + g[:, 1:2]


def _mlp_body(depth, with_cls):
    def body(x_ref, agg_ref, w0t_ref, wts_ref, b_ref, g_ref, be_ref, *rest):
        if with_cls:
            cwt_ref, cb_ref, out_ref = rest
        else:
            (out_ref,) = rest
        hin = x_ref[...] + agg_ref[...]
        h = jnp.transpose(hin)
        for i in range(depth):
            w = w0t_ref[...] if i == 0 else wts_ref[i - 1]
            h = jnp.dot(w, h, preferred_element_type=jnp.float32)
            h = h + b_ref[i]
            h = jnp.maximum(h, 0.0)
            mu = _sum_nodes(h) * _INV_N
            d = h - mu
            vs = _sum_nodes(d * d)
            s = jnp.sqrt(vs * _INV_N + _EPS)
            h = g_ref[i] * d / s + be_ref[i]
        h = jnp.maximum(h, 0.0)
        if with_cls:
            h = jnp.dot(cwt_ref[...], h, preferred_element_type=jnp.float32)
            h = h + cb_ref[...]
        out_ref[...] = jnp.transpose(h)

    return body


def _mlp_call(hin, agg, params, cls=None):
    depth = len(params)
    w0t = params[0][0].T
    wts = jnp.stack([params[i][0].T for i in range(1, depth)])
    bs = jnp.stack([p[1] for p in params])[:, :, None]
    gs = jnp.stack([p[2] for p in params])[:, :, None]
    bes = jnp.stack([p[3] for p in params])[:, :, None]
    args = [hin, agg, w0t, wts, bs, gs, bes]
    if cls is not None:
        args += [cls[0].T, cls[1].reshape(-1, 1)]
    out_ch = cls[0].shape[1] if cls is not None else params[-1][0].shape[1]
    return pl.pallas_call(
        _mlp_body(depth, cls is not None),
        out_shape=jax.ShapeDtypeStruct((hin.shape[0], out_ch), jnp.float32),
        compiler_params=pltpu.CompilerParams(
            vmem_limit_bytes=120 * 1024 * 1024,
        ),
    )(*args)


def kernel(x, edge_attr, edge_index, conv_params, cls_W, cls_b):
    del edge_attr
    src = edge_index[0]
    dst = edge_index[1]

    agg1 = jnp.zeros_like(x).at[dst].add(x[src])
    h1 = _mlp_call(x, agg1, conv_params[0])
    agg2 = jnp.zeros_like(h1).at[dst].add(h1[src])
    return _mlp_call(h1, agg2, conv_params[1], cls=(cls_W, cls_b))


# speedup vs baseline: 2.0893x; 2.0893x over previous
import functools

import jax
import jax.numpy as jnp
import numpy as np
from jax import lax
from jax.experimental import pallas as pl
from jax.experimental.pallas import tpu as pltpu
from jax.experimental.pallas import tpu_sc as plsc

N_NODES = 10000
N_EDGES = 320000
HID = 64
_NFULL = N_NODES // 128
_NTAIL = N_NODES - _NFULL * 128
_INV_N = float(np.float32(1e-4))
_EPS = float(np.float32(1e-5))

_CHUNK_LENS = ([10080] * 11 + [9840] * 4 + [9760]) * 2
_CHUNK_STARTS = [0]
for _l in _CHUNK_LENS[:-1]:
    _CHUNK_STARTS.append(_CHUNK_STARTS[-1] + _l)
_CPW = 10080
_K = 80
_NCH = _CPW // _K
_B = 912
_BREAL = _B - 1


def _sc_scatter(C):
    mesh = plsc.VectorSubcoreMesh(core_axis_name="c", subcore_axis_name="s")

    @functools.partial(
        pl.kernel,
        mesh=mesh,
        out_type=jax.ShapeDtypeStruct((32 * _B, C), jnp.float32),
        scratch_types=[
            pltpu.VMEM((_K,), jnp.int32),
            pltpu.VMEM((_K,), jnp.int32),
            pltpu.VMEM((_K, C), jnp.float32),
            pltpu.VMEM_SHARED((16 * _B, C), jnp.float32),
            pltpu.SemaphoreType.DMA,
        ],
    )
    def scatter_kernel(h_hbm, src_hbm, ldst_hbm, z_hbm, out_hbm,
                       idx_s, idx_d, rows, pacc, sem):
        cid = lax.axis_index("c")
        sid = lax.axis_index("s")
        w = cid * 16 + sid
        base = w * _CPW

        pltpu.sync_copy(z_hbm, pacc.at[pl.ds(sid * _B, _B)])

        def chunk(i, carry):
            off = base + i * _K
            pltpu.sync_copy(src_hbm.at[pl.ds(off, _K)], idx_s)
            pltpu.sync_copy(ldst_hbm.at[pl.ds(off, _K)], idx_d)
            pltpu.async_copy(h_hbm.at[idx_s], rows, sem).wait()
            pltpu.sync_copy(rows, pacc.at[idx_d], add=True)
            return carry

        lax.fori_loop(0, _NCH, chunk, 0)
        pltpu.sync_copy(pacc.at[pl.ds(sid * _B, _B)],
                        out_hbm.at[pl.ds(w * _B, _B)])

    return scatter_kernel


def _sum_nodes(h):
    acc = h[:, 0:128]
    for j in range(1, _NFULL):
        acc = acc + h[:, 128 * j:128 * (j + 1)]
    tail = h[:, _NFULL * 128:N_NODES]
    acc = jnp.concatenate([acc[:, 0:_NTAIL] + tail, acc[:, _NTAIL:]], axis=1)
    g = acc[:, 0:8]
    for k in range(1, 16):
        g = g + acc[:, 8 * k:8 * k + 8]
    g = g[:, 0:4] + g[:, 4:8]
    g = g[:, 0:2] + g[:, 2:4]
    return g[:, 0:1] + g[:, 1:2]


def _mlp_body(depth, with_cls, C_in, out_pad):
    def body(x_ref, blocks_ref, first_ref, w0t_ref, wts_ref, b_ref, g_ref,
             be_ref, *rest):
        if with_cls:
            cwt_ref, cb_ref, out_ref, acc_ref = rest
        else:
            out_ref, acc_ref = rest
        acc_ref[...] = jnp.zeros((N_NODES + _BREAL, C_in), jnp.float32)
        for w in range(32):
            r = first_ref[w]
            blk = blocks_ref[pl.ds(w * _B, _BREAL), 0:C_in]
            acc_ref[pl.ds(r, _BREAL), :] = acc_ref[pl.ds(r, _BREAL), :] + blk
        hin = x_ref[0:N_NODES, 0:C_in] + acc_ref[0:N_NODES, :]
        h = jnp.transpose(hin)
        for i in range(depth):
            wm = w0t_ref[...] if i == 0 else wts_ref[i - 1]
            h = jnp.dot(wm, h, preferred_element_type=jnp.float32)
            h = h + b_ref[i]
            h = jnp.maximum(h, 0.0)
            mu = _sum_nodes(h) * _INV_N
            d = h - mu
            vs = _sum_nodes(d * d)
            s = jnp.sqrt(vs * _INV_N + _EPS)
            h = g_ref[i] * d / s + be_ref[i]
        h = jnp.maximum(h, 0.0)
        if with_cls:
            h = jnp.dot(cwt_ref[...], h, preferred_element_type=jnp.float32)
            h = h + cb_ref[...]
        if out_pad:
            h = jnp.concatenate(
                [h, jnp.zeros((128 - h.shape[0], N_NODES), jnp.float32)], 0)
        out_ref[...] = jnp.transpose(h)

    return body


def _mlp_call(hin, blocks, first, params, cls=None, C_in=None, out_pad=False):
    depth = len(params)
    if C_in is None:
        C_in = hin.shape[1]
    w0t = params[0][0].T
    wts = jnp.stack([params[i][0].T for i in range(1, depth)])
    bs = jnp.stack([p[1] for p in params])[:, :, None]
    gs = jnp.stack([p[2] for p in params])[:, :, None]
    bes = jnp.stack([p[3] for p in params])[:, :, None]
    args = [hin, blocks, first, w0t, wts, bs, gs, bes]
    if cls is not None:
        args += [cls[0].T, cls[1].reshape(-1, 1)]
    out_ch = cls[0].shape[1] if cls is not None else params[-1][0].shape[1]
    if out_pad:
        out_ch = 128
    n_in = len(args)
    in_specs = [pl.BlockSpec(memory_space=pltpu.VMEM) for _ in range(n_in)]
    in_specs[2] = pl.BlockSpec(memory_space=pltpu.SMEM)
    return pl.pallas_call(
        _mlp_body(depth, cls is not None, C_in, out_pad),
        out_shape=jax.ShapeDtypeStruct((hin.shape[0], out_ch), jnp.float32),
        in_specs=in_specs,
        out_specs=pl.BlockSpec(memory_space=pltpu.VMEM),
        scratch_shapes=[pltpu.VMEM((N_NODES + _BREAL, C_in), jnp.float32)],
        compiler_params=pltpu.CompilerParams(
            vmem_limit_bytes=128 * 1024 * 1024,
        ),
    )(*args)


def kernel(x, edge_attr, edge_index, conv_params, cls_W, cls_b):
    del edge_attr
    src = edge_index[0].astype(jnp.int32)
    dst = edge_index[1].astype(jnp.int32)

    order = jnp.argsort(dst, stable=True)
    ssrc = src[order]
    sdst = dst[order]
    first_list, src_parts, ldst_parts = [], [], []
    for k in range(32):
        s, ln = _CHUNK_STARTS[k], _CHUNK_LENS[k]
        f = sdst[s]
        first_list.append(f)
        pad = _CPW - ln
        src_parts.append(jnp.pad(ssrc[s:s + ln], (0, pad)))
        ld = jnp.clip(sdst[s:s + ln] - f, 0, _BREAL - 1)
        ld2 = (k % 16) * _B + ld
        ldst_parts.append(jnp.pad(ld2, (0, pad),
                                  constant_values=(k % 16) * _B + _BREAL))
    srcp = jnp.concatenate(src_parts)
    ldstp = jnp.concatenate(ldst_parts)
    first = jnp.stack(first_list)

    def agg_blocks(h):
        C = h.shape[1]
        z = jnp.zeros((_B, C), jnp.float32)
        return _sc_scatter(C)(h, srcp, ldstp, z)

    b1 = agg_blocks(x)
    h1p = _mlp_call(x, b1, first, conv_params[0], out_pad=True)
    b2 = agg_blocks(h1p)
    return _mlp_call(h1p, b2, first, conv_params[1], cls=(cls_W, cls_b),
                     C_in=HID)
